# baseline (speedup 1.0000x reference)
import functools

import jax
import jax.numpy as jnp
from jax.experimental import pallas as pl
from jax.experimental.pallas import tpu as pltpu

_EPS = 1e-5
_C_IN = 3
_KH = _KW = 3
_K = _C_IN * _KH * _KW
_G = 8


def _im2col_stats_kernel(x_ref, t_ref, l_ref, out_ref, psum_ref, psq_ref,
                         *, ho, wo, n_strips, out_depth):
    strip = pl.program_id(1)
    t = t_ref[...]
    for c in range(_C_IN):
        xe = x_ref[0, c, pl.ds(0, ho, 2), :].astype(jnp.bfloat16)
        xo = x_ref[0, c, pl.ds(1, ho, 2), :].astype(jnp.bfloat16)
        de = jnp.dot(xe, t, preferred_element_type=jnp.float32)
        do = jnp.dot(xo, t, preferred_element_type=jnp.float32)
        dm = jnp.concatenate(
            [jnp.zeros((1, _KW * wo), jnp.float32), do[:ho - 1]], axis=0)
        for kh, src in ((0, dm), (1, de), (2, do)):
            for kw in range(_KW):
                k = c * _KH * _KW + kh * _KW + kw
                plane = src[:, kw * wo:(kw + 1) * wo].astype(jnp.bfloat16)

                @pl.when(strip == 0)
                def _(k=k, plane=plane):
                    out_ref[0, k] = plane

                @pl.when(strip != 0)
                def _(k=k, plane=plane):
                    out_ref[0, k] = out_ref[0, k] + plane

    @pl.when(strip == n_strips - 1)
    def _():
        l = l_ref[...]
        zpad = jnp.zeros((2 * 128 - _K * _G, wo), jnp.bfloat16)
        s_conv = jnp.zeros((16, wo), jnp.float32)
        q_conv = jnp.zeros((16, wo), jnp.float32)
        for g in range(ho // _G):
            rhs = out_ref[0, :, pl.ds(g * _G, _G), :]
            rhs = jnp.concatenate([rhs.reshape(_K * _G, wo), zpad], axis=0)
            stk = jnp.dot(l, rhs, preferred_element_type=jnp.float32)
            rs = stk.reshape(16, _G, wo)
            s_conv = s_conv + jnp.sum(rs, axis=1)
            q_conv = q_conv + jnp.sum(rs * rs, axis=1)
        s_col = jnp.sum(s_conv, axis=1, keepdims=True)
        q_col = jnp.sum(q_conv, axis=1, keepdims=True)
        pool_s, pool_q = [], []
        for c in range(_C_IN):
            b = c * _KH * _KW
            p = jnp.maximum(
                jnp.maximum(out_ref[0, b + 4], out_ref[0, b + 5]),
                jnp.maximum(out_ref[0, b + 7], out_ref[0, b + 8]),
            ).astype(jnp.float32)
            pool_s.append(jnp.sum(jnp.sum(p, axis=0, keepdims=True),
                                  axis=1, keepdims=True))
            pool_q.append(jnp.sum(jnp.sum(p * p, axis=0, keepdims=True),
                                  axis=1, keepdims=True))
        psum_ref[0] = jnp.concatenate([s_col[:out_depth]] + pool_s, axis=0)
        psq_ref[0] = jnp.concatenate([q_col[:out_depth]] + pool_q, axis=0)


def _im2col_pallas(x, l_mat, ho, wo, out_depth, c_total):
    n, _, h, w_in = x.shape
    n_strips = w_in // 128
    j = jax.lax.broadcasted_iota(jnp.int32, (w_in, _KW * wo), 0)
    col = jax.lax.broadcasted_iota(jnp.int32, (w_in, _KW * wo), 1)
    sel = 2 * (col % wo) + col // wo - 1
    t = (j == sel).astype(jnp.bfloat16)

    kern = functools.partial(_im2col_stats_kernel, ho=ho, wo=wo,
                             n_strips=n_strips, out_depth=out_depth)
    return pl.pallas_call(
        kern,
        out_shape=(jax.ShapeDtypeStruct((n, _K, ho, wo), jnp.bfloat16),
                   jax.ShapeDtypeStruct((n, c_total, 1), jnp.float32),
                   jax.ShapeDtypeStruct((n, c_total, 1), jnp.float32)),
        grid_spec=pltpu.PrefetchScalarGridSpec(
            num_scalar_prefetch=0,
            grid=(n, n_strips),
            in_specs=[pl.BlockSpec((1, _C_IN, h, 128),
                                   lambda i, st: (i, 0, 0, st)),
                      pl.BlockSpec((128, _KW * wo), lambda i, st: (st, 0)),
                      pl.BlockSpec((c_total * _G, 2 * 128),
                                   lambda i, st: (0, 0))],
            out_specs=[pl.BlockSpec((1, _K, ho, wo),
                                    lambda i, st: (i, 0, 0, 0)),
                       pl.BlockSpec((1, c_total, 1), lambda i, st: (i, 0, 0)),
                       pl.BlockSpec((1, c_total, 1),
                                    lambda i, st: (i, 0, 0))]),
        compiler_params=pltpu.CompilerParams(
            dimension_semantics=("parallel", "arbitrary"),
            vmem_limit_bytes=64 * 1024 * 1024),
    )(x, t, l_mat)


def _conv_groups(p4, l_ref, rb, wo):
    l = l_ref[...]
    zpad = jnp.zeros((2 * 128 - _K * _G, wo), jnp.bfloat16)
    outs = []
    for g in range(rb // _G):
        rhs = p4[:, g * _G:(g + 1) * _G, :].reshape(_K * _G, wo)
        rhs = jnp.concatenate([rhs, zpad], axis=0)
        stk = jnp.dot(l, rhs, preferred_element_type=jnp.float32)
        outs.append(stk.reshape(16, _G, wo))
    return jnp.concatenate(outs, axis=1)


def _pool3(p4):
    pools = []
    for c in range(_C_IN):
        b = c * _KH * _KW
        m0 = jnp.maximum(p4[b + 4], p4[b + 5])
        m1 = jnp.maximum(p4[b + 7], p4[b + 8])
        pools.append(jnp.maximum(m0, m1)[None])
    return jnp.concatenate(pools, axis=0).astype(jnp.float32)


def _out_kernel(p_ref, l_ref, scale_ref, shift_ref, alpha_ref, out_ref,
                *, rb, wo, out_depth):
    p4 = p_ref[0]
    conv = _conv_groups(p4, l_ref, rb, wo)
    raw = jnp.concatenate([conv[:out_depth], _pool3(p4)], axis=0)
    y = raw * scale_ref[...][:, :, None] + shift_ref[...][:, :, None]
    out_ref[0] = jnp.where(y >= 0.0, y, alpha_ref[...][:, :, None] * y)


def _initial_block(x, conv_w, gamma, beta, alpha):
    N, c_in, H, W = x.shape
    assert c_in == _C_IN
    out_depth = conv_w.shape[0]
    c_total = out_depth + _C_IN
    Ho, Wo = H // 2, W // 2
    M = N * Ho * Wo

    rb = _G
    for cand_rb in (256, 128, 64, 32, 16):
        if Ho % cand_rb == 0:
            rb = cand_rb
            break
    assert Wo % 128 == 0 and Ho % rb == 0, "unsupported shape"
    tpi = Ho // rb
    n_tiles = N * tpi

    w_mat = jnp.pad(conv_w.reshape(out_depth, _K), ((0, c_total - out_depth),
                                                    (0, 0)))
    l_mat = jnp.kron(w_mat, jnp.eye(_G, dtype=w_mat.dtype))
    l_mat = jnp.pad(l_mat, ((0, 0), (0, 2 * 128 - _K * _G)))
    l_mat = l_mat.astype(jnp.bfloat16)

    patches, psum, psq = _im2col_pallas(x, l_mat, Ho, Wo, out_depth, c_total)

    ssum = jnp.sum(psum[:, :, 0], axis=0)
    ssq = jnp.sum(psq[:, :, 0], axis=0)
    mean = ssum / M
    var = jnp.maximum(ssq / M - mean * mean, 0.0)
    inv = jax.lax.rsqrt(var + _EPS)
    g = gamma.astype(jnp.float32)
    scale = (g * inv).reshape(c_total, 1)
    shift = (beta.astype(jnp.float32) - mean * g * inv).reshape(c_total, 1)
    alpha_col = jnp.broadcast_to(jnp.asarray(alpha, jnp.float32), (c_total, 1))

    kern_b = functools.partial(_out_kernel, rb=rb, wo=Wo, out_depth=out_depth)
    y = pl.pallas_call(
        kern_b,
        out_shape=jax.ShapeDtypeStruct((N, c_total, Ho, Wo), jnp.float32),
        grid_spec=pltpu.PrefetchScalarGridSpec(
            num_scalar_prefetch=0,
            grid=(n_tiles,),
            in_specs=[pl.BlockSpec((1, _K, rb, Wo),
                                   lambda i: (i // tpi, 0, i % tpi, 0)),
                      pl.BlockSpec((c_total * _G, 2 * 128),
                                   lambda i: (0, 0)),
                      pl.BlockSpec((c_total, 1), lambda i: (0, 0)),
                      pl.BlockSpec((c_total, 1), lambda i: (0, 0)),
                      pl.BlockSpec((c_total, 1), lambda i: (0, 0))],
            out_specs=pl.BlockSpec((1, c_total, rb, Wo),
                                   lambda i: (i // tpi, 0, i % tpi, 0))),
        compiler_params=pltpu.CompilerParams(
            dimension_semantics=("parallel",),
            vmem_limit_bytes=64 * 1024 * 1024),
    )(patches, l_mat, scale, shift, alpha_col)

    return y


def kernel(x, conv_w, gamma, beta, alpha):
    return _initial_block(x, conv_w, gamma, beta, alpha)


# speedup vs baseline: 1.3232x; 1.3232x over previous
import functools

import jax
import jax.numpy as jnp
from jax.experimental import pallas as pl
from jax.experimental.pallas import tpu as pltpu

_EPS = 1e-5
_C_IN = 3
_KH = _KW = 3
_K = _C_IN * _KH * _KW
_G = 8


def _im2col_stats_kernel(x_ref, t_ref, l_ref, out_ref, psum_ref, psq_ref,
                         *, ho, wo, out_depth):
    t = t_ref[...]
    w_in = t.shape[0]
    pool_parts = []
    for c in range(_C_IN):
        xe = x_ref[0, c, :, :w_in].astype(jnp.bfloat16)
        xo = x_ref[0, c, :, w_in:].astype(jnp.bfloat16)
        de = jnp.dot(xe, t, preferred_element_type=jnp.float32)
        do = jnp.dot(xo, t, preferred_element_type=jnp.float32)
        dm = jnp.concatenate(
            [jnp.zeros((1, _KW * wo), jnp.float32), do[:ho - 1]], axis=0)
        for kh, src in ((0, dm), (1, de), (2, do)):
            for kw in range(_KW):
                out_ref[0, c * _KH * _KW + kh * _KW + kw] = (
                    src[:, kw * wo:(kw + 1) * wo].astype(jnp.bfloat16))
        pool_c = jnp.maximum(jnp.maximum(de[:, wo:2 * wo], de[:, 2 * wo:]),
                             jnp.maximum(do[:, wo:2 * wo], do[:, 2 * wo:]))
        pool_parts.append(pool_c)

    l = l_ref[...]
    zpad = jnp.zeros((2 * 128 - _K * _G, wo), jnp.bfloat16)
    s_conv = jnp.zeros((16, wo), jnp.float32)
    q_conv = jnp.zeros((16, wo), jnp.float32)
    for g in range(ho // _G):
        rhs = out_ref[0, :, pl.ds(g * _G, _G), :]
        rhs = jnp.concatenate([rhs.reshape(_K * _G, wo), zpad], axis=0)
        stk = jnp.dot(l, rhs, preferred_element_type=jnp.float32)
        rs = stk.reshape(16, _G, wo)
        s_conv = s_conv + jnp.sum(rs, axis=1)
        q_conv = q_conv + jnp.sum(rs * rs, axis=1)
    s_col = jnp.sum(s_conv, axis=1, keepdims=True)
    q_col = jnp.sum(q_conv, axis=1, keepdims=True)
    pool_s = jnp.concatenate(
        [jnp.sum(jnp.sum(p, axis=0, keepdims=True), axis=1, keepdims=True)
         for p in pool_parts], axis=0)
    pool_q = jnp.concatenate(
        [jnp.sum(jnp.sum(p * p, axis=0, keepdims=True), axis=1, keepdims=True)
         for p in pool_parts], axis=0)
    psum_ref[0] = jnp.concatenate([s_col[:out_depth], pool_s], axis=0)
    psq_ref[0] = jnp.concatenate([q_col[:out_depth], pool_q], axis=0)


def _im2col_pallas(x, l_mat, ho, wo, out_depth, c_total):
    n, _, h, w_in = x.shape
    x4 = x.reshape(n, _C_IN, ho, 2 * w_in)
    j = jax.lax.broadcasted_iota(jnp.int32, (w_in, _KW * wo), 0)
    col = jax.lax.broadcasted_iota(jnp.int32, (w_in, _KW * wo), 1)
    sel = 2 * (col % wo) + col // wo - 1
    t = (j == sel).astype(jnp.bfloat16)

    kern = functools.partial(_im2col_stats_kernel, ho=ho, wo=wo,
                             out_depth=out_depth)
    return pl.pallas_call(
        kern,
        out_shape=(jax.ShapeDtypeStruct((n, _K, ho, wo), jnp.bfloat16),
                   jax.ShapeDtypeStruct((n, c_total, 1), jnp.float32),
                   jax.ShapeDtypeStruct((n, c_total, 1), jnp.float32)),
        grid_spec=pltpu.PrefetchScalarGridSpec(
            num_scalar_prefetch=0,
            grid=(n,),
            in_specs=[pl.BlockSpec((1, _C_IN, ho, 2 * w_in),
                                   lambda i: (i, 0, 0, 0)),
                      pl.BlockSpec((w_in, _KW * wo), lambda i: (0, 0)),
                      pl.BlockSpec((c_total * _G, 2 * 128),
                                   lambda i: (0, 0))],
            out_specs=[pl.BlockSpec((1, _K, ho, wo), lambda i: (i, 0, 0, 0)),
                       pl.BlockSpec((1, c_total, 1), lambda i: (i, 0, 0)),
                       pl.BlockSpec((1, c_total, 1), lambda i: (i, 0, 0))]),
        compiler_params=pltpu.CompilerParams(
            dimension_semantics=("parallel",),
            vmem_limit_bytes=64 * 1024 * 1024),
    )(x4, t, l_mat)


def _conv_groups(p4, l_ref, rb, wo):
    l = l_ref[...]
    zpad = jnp.zeros((2 * 128 - _K * _G, wo), jnp.bfloat16)
    outs = []
    for g in range(rb // _G):
        rhs = p4[:, g * _G:(g + 1) * _G, :].reshape(_K * _G, wo)
        rhs = jnp.concatenate([rhs, zpad], axis=0)
        stk = jnp.dot(l, rhs, preferred_element_type=jnp.float32)
        outs.append(stk.reshape(16, _G, wo))
    return jnp.concatenate(outs, axis=1)


def _pool3(p4):
    pools = []
    for c in range(_C_IN):
        b = c * _KH * _KW
        m0 = jnp.maximum(p4[b + 4], p4[b + 5])
        m1 = jnp.maximum(p4[b + 7], p4[b + 8])
        pools.append(jnp.maximum(m0, m1)[None])
    return jnp.concatenate(pools, axis=0).astype(jnp.float32)


def _out_kernel(p_ref, l_ref, scale_ref, shift_ref, alpha_ref, out_ref,
                *, rb, wo, out_depth):
    p4 = p_ref[0]
    conv = _conv_groups(p4, l_ref, rb, wo)
    raw = jnp.concatenate([conv[:out_depth], _pool3(p4)], axis=0)
    y = raw * scale_ref[...][:, :, None] + shift_ref[...][:, :, None]
    out_ref[0] = jnp.where(y >= 0.0, y, alpha_ref[...][:, :, None] * y)


def _initial_block(x, conv_w, gamma, beta, alpha):
    N, c_in, H, W = x.shape
    assert c_in == _C_IN
    out_depth = conv_w.shape[0]
    c_total = out_depth + _C_IN
    Ho, Wo = H // 2, W // 2
    M = N * Ho * Wo

    rb = _G
    for cand_rb in (256, 128, 64, 32, 16):
        if Ho % cand_rb == 0:
            rb = cand_rb
            break
    assert Wo % 128 == 0 and Ho % rb == 0, "unsupported shape"
    tpi = Ho // rb
    n_tiles = N * tpi

    w_mat = jnp.pad(conv_w.reshape(out_depth, _K), ((0, c_total - out_depth),
                                                    (0, 0)))
    l_mat = jnp.kron(w_mat, jnp.eye(_G, dtype=w_mat.dtype))
    l_mat = jnp.pad(l_mat, ((0, 0), (0, 2 * 128 - _K * _G)))
    l_mat = l_mat.astype(jnp.bfloat16)

    patches, psum, psq = _im2col_pallas(x, l_mat, Ho, Wo, out_depth, c_total)

    ssum = jnp.sum(psum[:, :, 0], axis=0)
    ssq = jnp.sum(psq[:, :, 0], axis=0)
    mean = ssum / M
    var = jnp.maximum(ssq / M - mean * mean, 0.0)
    inv = jax.lax.rsqrt(var + _EPS)
    g = gamma.astype(jnp.float32)
    scale = (g * inv).reshape(c_total, 1)
    shift = (beta.astype(jnp.float32) - mean * g * inv).reshape(c_total, 1)
    alpha_col = jnp.broadcast_to(jnp.asarray(alpha, jnp.float32), (c_total, 1))

    kern_b = functools.partial(_out_kernel, rb=rb, wo=Wo, out_depth=out_depth)
    y = pl.pallas_call(
        kern_b,
        out_shape=jax.ShapeDtypeStruct((N, c_total, Ho, Wo), jnp.float32),
        grid_spec=pltpu.PrefetchScalarGridSpec(
            num_scalar_prefetch=0,
            grid=(n_tiles,),
            in_specs=[pl.BlockSpec((1, _K, rb, Wo),
                                   lambda i: (i // tpi, 0, i % tpi, 0)),
                      pl.BlockSpec((c_total * _G, 2 * 128),
                                   lambda i: (0, 0)),
                      pl.BlockSpec((c_total, 1), lambda i: (0, 0)),
                      pl.BlockSpec((c_total, 1), lambda i: (0, 0)),
                      pl.BlockSpec((c_total, 1), lambda i: (0, 0))],
            out_specs=pl.BlockSpec((1, c_total, rb, Wo),
                                   lambda i: (i // tpi, 0, i % tpi, 0))),
        compiler_params=pltpu.CompilerParams(
            dimension_semantics=("parallel",),
            vmem_limit_bytes=64 * 1024 * 1024),
    )(patches, l_mat, scale, shift, alpha_col)

    return y


def kernel(x, conv_w, gamma, beta, alpha):
    return _initial_block(x, conv_w, gamma, beta, alpha)
